# baseline (speedup 1.0000x reference)
import functools

import jax
import jax.numpy as jnp
from jax import lax
from jax.experimental import pallas as pl
from jax.experimental.pallas import tpu as pltpu
from jax.experimental.pallas import tpu_sc as plsc

COMP_RATIO = 0.5


def _reduce_select_body(nqb, k, ch, a_ref, idx_ref, acc_ref):
    i = pl.program_id(0)
    mh = jnp.sum(a_ref[...], axis=0) / a_ref.shape[0]
    part = jnp.sum(mh, axis=0, keepdims=True)

    @pl.when(i == 0)
    def _init():
        acc_ref[...] = jnp.zeros_like(acc_ref)

    acc_ref[...] += part

    @pl.when(i == nqb - 1)
    def _fini():
        s_row = acc_ref[...] / float(a_ref.shape[1] * nqb)
        _select_block(s_row, k, ch, idx_ref)


def _reduce_select(att, k):
    h, s, _ = att.shape
    qb = 128
    nqb = s // qb
    return pl.pallas_call(
        functools.partial(_reduce_select_body, nqb, k, 256),
        grid=(nqb,),
        in_specs=[pl.BlockSpec((h, qb, s), lambda i: (0, i, 0))],
        out_specs=pl.BlockSpec((1, k), lambda i: (0, 0)),
        out_shape=jax.ShapeDtypeStruct((1, k), jnp.int32),
        scratch_shapes=[pltpu.VMEM((1, s), jnp.float32)],
    )(att)


def _select_block(s_row, k, ch, idx_ref):
    seq = s_row.shape[1]
    nch = seq // ch
    lane = lax.broadcasted_iota(jnp.int32, (ch, seq), 1)
    row = lax.broadcasted_iota(jnp.int32, (ch, seq), 0)
    gcol = lax.broadcasted_iota(jnp.int32, (ch, 1), 0).astype(jnp.float32)
    slot = lax.broadcasted_iota(jnp.int32, (1, k), 1).astype(jnp.float32)
    r_tri = lax.broadcasted_iota(jnp.int32, (ch, ch), 0)
    c_tri = lax.broadcasted_iota(jnp.int32, (ch, ch), 1)
    tri = (c_tri <= r_tri).astype(jnp.float32)

    acc = jnp.zeros((1, k), jnp.float32)
    carry = jnp.float32(0.0)
    for c in range(nch):
        base = c * ch
        gi = row + base
        ident = (lane == gi).astype(jnp.float32)
        s_col = lax.dot_general(ident, s_row,
                                (((1,), (1,)), ((), ())),
                                precision=lax.Precision.HIGHEST)
        gt = (s_row > s_col).astype(jnp.float32)
        eq_lt = ((s_row == s_col) & (lane < gi)).astype(jnp.float32)
        rank = jnp.sum(gt + eq_lt, axis=1, keepdims=True)
        m_col = (rank < float(k)).astype(jnp.float32)
        incl = lax.dot_general(tri, m_col,
                               (((1,), (0,)), ((), ())),
                               precision=lax.Precision.HIGHEST)
        dest = incl - 1.0 + carry
        carry = carry + jnp.sum(m_col)
        onehot = (dest == slot).astype(jnp.float32) * m_col
        acc = acc + lax.dot_general(gcol + float(base), onehot,
                                    (((0,), (0,)), ((), ())),
                                    precision=lax.Precision.HIGHEST)
    idx_ref[...] = acc.astype(jnp.int32)


_NC, _NS = 2, 16
_NW = _NC * _NS


def _gather_sc_body(rpw, emb_hbm, idx_hbm, out_hbm, idx_v, rows_v, sem):
    wid = lax.axis_index("s") * _NC + lax.axis_index("c")
    base = wid * rpw
    pltpu.sync_copy(idx_hbm.at[pl.ds(base, rpw)], idx_v)
    pltpu.async_copy(emb_hbm.at[idx_v], rows_v, sem).wait()
    pltpu.sync_copy(rows_v, out_hbm.at[pl.ds(base, rpw)])


def _gather(emb2d, idx1d):
    k = idx1d.shape[0]
    d = emb2d.shape[1]
    rpw = k // _NW
    mesh = plsc.VectorSubcoreMesh(core_axis_name="c", subcore_axis_name="s")
    return pl.kernel(
        functools.partial(_gather_sc_body, rpw),
        out_type=jax.ShapeDtypeStruct((k, d), jnp.float32),
        mesh=mesh,
        scratch_types=[
            pltpu.VMEM((rpw,), jnp.int32),
            pltpu.VMEM((rpw, d), jnp.float32),
            pltpu.SemaphoreType.DMA,
        ],
    )(emb2d, idx1d)


def kernel(context_embeddings, attention_weights):
    b, s, d = context_embeddings.shape
    h = attention_weights.shape[1]
    k = max(1, int(s * COMP_RATIO))
    emb = context_embeddings.reshape(s, d)
    att = attention_weights.reshape(h, s, s)
    idx = _reduce_select(att, k)
    out = _gather(emb, idx.reshape(k))
    return out.reshape(b, k, d)


# speedup vs baseline: 1.0569x; 1.0569x over previous
import functools

import jax
import jax.numpy as jnp
from jax import lax
from jax.experimental import pallas as pl
from jax.experimental.pallas import tpu as pltpu
from jax.experimental.pallas import tpu_sc as plsc

COMP_RATIO = 0.5


def _reduce_select_body(nqb, k, ch, a_ref, idx_ref, acc_ref):
    i = pl.program_id(0)
    mh = jnp.sum(a_ref[...], axis=0) / a_ref.shape[0]
    part = jnp.sum(mh, axis=0, keepdims=True)

    @pl.when(i == 0)
    def _init():
        acc_ref[...] = jnp.zeros_like(acc_ref)

    acc_ref[...] += part

    @pl.when(i == nqb - 1)
    def _fini():
        s_row = acc_ref[...] / float(a_ref.shape[1] * nqb)
        _select_block(s_row, k, ch, idx_ref)


def _reduce_select(att, k):
    h, s, _ = att.shape
    qb = 128
    nqb = s // qb
    return pl.pallas_call(
        functools.partial(_reduce_select_body, nqb, k, 256),
        grid=(nqb,),
        in_specs=[pl.BlockSpec((h, qb, s), lambda i: (0, i, 0))],
        out_specs=pl.BlockSpec((1, k), lambda i: (0, 0)),
        out_shape=jax.ShapeDtypeStruct((1, k), jnp.int32),
        scratch_shapes=[pltpu.VMEM((1, s), jnp.float32)],
    )(att)


def _select_block(s_row, k, ch, idx_ref):
    seq = s_row.shape[1]
    nch = seq // ch
    lane = lax.broadcasted_iota(jnp.int32, (ch, seq), 1)
    row = lax.broadcasted_iota(jnp.int32, (ch, seq), 0)
    gcol = lax.broadcasted_iota(jnp.int32, (ch, 1), 0).astype(jnp.float32)
    slot = lax.broadcasted_iota(jnp.int32, (1, k), 1).astype(jnp.float32)
    r_tri = lax.broadcasted_iota(jnp.int32, (ch, ch), 0)
    c_tri = lax.broadcasted_iota(jnp.int32, (ch, ch), 1)
    tri = (c_tri <= r_tri).astype(jnp.float32)

    acc = jnp.zeros((1, k), jnp.float32)
    carry = jnp.float32(0.0)
    for c in range(nch):
        base = c * ch
        gi = row + base
        ident = (lane == gi).astype(jnp.float32)
        s_col = lax.dot_general(ident, s_row,
                                (((1,), (1,)), ((), ())),
                                precision=lax.Precision.HIGHEST)
        gt = (s_row > s_col).astype(jnp.float32)
        eq_lt = ((s_row == s_col) & (lane < gi)).astype(jnp.float32)
        rank = jnp.sum(gt + eq_lt, axis=1, keepdims=True)
        m_col = (rank < float(k)).astype(jnp.float32)
        incl = lax.dot_general(tri, m_col,
                               (((1,), (0,)), ((), ())))
        dest = incl - 1.0 + carry
        carry = carry + jnp.sum(m_col)
        onehot = (dest == slot).astype(jnp.float32) * m_col
        acc = (acc
               + lax.dot_general(gcol, onehot, (((0,), (0,)), ((), ())))
               + float(base) * jnp.sum(onehot, axis=0, keepdims=True))
    idx_ref[...] = acc.astype(jnp.int32)


_NC, _NS = 2, 16
_NW = _NC * _NS


def _gather_sc_body(rpw, emb_hbm, idx_hbm, out_hbm, idx_v, rows_v, sem):
    wid = lax.axis_index("s") * _NC + lax.axis_index("c")
    base = wid * rpw
    pltpu.sync_copy(idx_hbm.at[pl.ds(base, rpw)], idx_v)
    pltpu.async_copy(emb_hbm.at[idx_v], rows_v, sem).wait()
    pltpu.sync_copy(rows_v, out_hbm.at[pl.ds(base, rpw)])


def _gather(emb2d, idx1d):
    k = idx1d.shape[0]
    d = emb2d.shape[1]
    rpw = k // _NW
    mesh = plsc.VectorSubcoreMesh(core_axis_name="c", subcore_axis_name="s")
    return pl.kernel(
        functools.partial(_gather_sc_body, rpw),
        out_type=jax.ShapeDtypeStruct((k, d), jnp.float32),
        mesh=mesh,
        scratch_types=[
            pltpu.VMEM((rpw,), jnp.int32),
            pltpu.VMEM((rpw, d), jnp.float32),
            pltpu.SemaphoreType.DMA,
        ],
    )(emb2d, idx1d)


def kernel(context_embeddings, attention_weights):
    b, s, d = context_embeddings.shape
    h = attention_weights.shape[1]
    k = max(1, int(s * COMP_RATIO))
    emb = context_embeddings.reshape(s, d)
    att = attention_weights.reshape(h, s, s)
    idx = _reduce_select(att, k)
    out = _gather(emb, idx.reshape(k))
    return out.reshape(b, k, d)


# speedup vs baseline: 1.0603x; 1.0032x over previous
import functools

import jax
import jax.numpy as jnp
from jax import lax
from jax.experimental import pallas as pl
from jax.experimental.pallas import tpu as pltpu
from jax.experimental.pallas import tpu_sc as plsc

COMP_RATIO = 0.5


def _reduce_select_body(nqb, k, ch, a_ref, idx_ref, acc_ref):
    i = pl.program_id(0)
    mh = jnp.sum(a_ref[...], axis=0) / a_ref.shape[0]
    part = jnp.sum(mh, axis=0, keepdims=True)

    @pl.when(i == 0)
    def _init():
        acc_ref[...] = jnp.zeros_like(acc_ref)

    acc_ref[...] += part

    @pl.when(i == nqb - 1)
    def _fini():
        s_row = acc_ref[...] / float(a_ref.shape[1] * nqb)
        _select_block(s_row, k, ch, idx_ref)


def _reduce_select(att, k):
    h, s, _ = att.shape
    qb = 64
    nqb = s // qb
    return pl.pallas_call(
        functools.partial(_reduce_select_body, nqb, k, 256),
        grid=(nqb,),
        in_specs=[pl.BlockSpec((h, qb, s), lambda i: (0, i, 0))],
        out_specs=pl.BlockSpec((1, k), lambda i: (0, 0)),
        out_shape=jax.ShapeDtypeStruct((1, k), jnp.int32),
        scratch_shapes=[pltpu.VMEM((1, s), jnp.float32)],
    )(att)


def _select_block(s_row, k, ch, idx_ref):
    seq = s_row.shape[1]
    nch = seq // ch
    lane = lax.broadcasted_iota(jnp.int32, (ch, seq), 1)
    row = lax.broadcasted_iota(jnp.int32, (ch, seq), 0)
    gcol = lax.broadcasted_iota(jnp.int32, (ch, 1), 0).astype(jnp.float32)
    slot = lax.broadcasted_iota(jnp.int32, (1, k), 1).astype(jnp.float32)
    r_tri = lax.broadcasted_iota(jnp.int32, (ch, ch), 0)
    c_tri = lax.broadcasted_iota(jnp.int32, (ch, ch), 1)
    tri = (c_tri <= r_tri).astype(jnp.float32)

    acc = jnp.zeros((1, k), jnp.float32)
    carry = jnp.float32(0.0)
    for c in range(nch):
        base = c * ch
        gi = row + base
        ident = (lane == gi).astype(jnp.float32)
        s_col = lax.dot_general(ident, s_row,
                                (((1,), (1,)), ((), ())),
                                precision=lax.Precision.HIGHEST)
        gt = (s_row > s_col).astype(jnp.float32)
        eq_lt = ((s_row == s_col) & (lane < gi)).astype(jnp.float32)
        rank = jnp.sum(gt + eq_lt, axis=1, keepdims=True)
        m_col = (rank < float(k)).astype(jnp.float32)
        incl = lax.dot_general(tri, m_col,
                               (((1,), (0,)), ((), ())))
        dest = incl - 1.0 + carry
        carry = carry + jnp.sum(m_col)
        onehot = (dest == slot).astype(jnp.float32) * m_col
        acc = (acc
               + lax.dot_general(gcol, onehot, (((0,), (0,)), ((), ())))
               + float(base) * jnp.sum(onehot, axis=0, keepdims=True))
    idx_ref[...] = acc.astype(jnp.int32)


_NC, _NS = 2, 16
_NW = _NC * _NS


def _gather_sc_body(rpw, emb_hbm, idx_hbm, out_hbm, idx_v, rows_v, sem):
    wid = lax.axis_index("s") * _NC + lax.axis_index("c")
    base = wid * rpw
    pltpu.sync_copy(idx_hbm.at[pl.ds(base, rpw)], idx_v)
    pltpu.async_copy(emb_hbm.at[idx_v], rows_v, sem).wait()
    pltpu.sync_copy(rows_v, out_hbm.at[pl.ds(base, rpw)])


def _gather(emb2d, idx1d):
    k = idx1d.shape[0]
    d = emb2d.shape[1]
    rpw = k // _NW
    mesh = plsc.VectorSubcoreMesh(core_axis_name="c", subcore_axis_name="s")
    return pl.kernel(
        functools.partial(_gather_sc_body, rpw),
        out_type=jax.ShapeDtypeStruct((k, d), jnp.float32),
        mesh=mesh,
        scratch_types=[
            pltpu.VMEM((rpw,), jnp.int32),
            pltpu.VMEM((rpw, d), jnp.float32),
            pltpu.SemaphoreType.DMA,
        ],
    )(emb2d, idx1d)


def kernel(context_embeddings, attention_weights):
    b, s, d = context_embeddings.shape
    h = attention_weights.shape[1]
    k = max(1, int(s * COMP_RATIO))
    emb = context_embeddings.reshape(s, d)
    att = attention_weights.reshape(h, s, s)
    idx = _reduce_select(att, k)
    out = _gather(emb, idx.reshape(k))
    return out.reshape(b, k, d)
